# Initial kernel scaffold; baseline (speedup 1.0000x reference)
import functools

import jax
import jax.numpy as jnp
from jax import lax
from jax.experimental import pallas as pl
from jax.experimental.pallas import tpu as pltpu
from jax.experimental.pallas import tpu_sc as plsc

_NUM_WORKERS = 32
_CHUNK = 512


@functools.lru_cache(maxsize=None)
def _make_kernel(B, D, chunk):
    b_per_w = B // _NUM_WORKERS
    n_chunks = b_per_w // chunk
    mesh = plsc.VectorSubcoreMesh(core_axis_name="c", subcore_axis_name="s")

    @functools.partial(
        pl.kernel,
        mesh=mesh,
        out_type=(
            jax.ShapeDtypeStruct((B, D), jnp.float32),
            jax.ShapeDtypeStruct((B, D), jnp.float32),
        ),
        scratch_types=[
            pltpu.VMEM((chunk,), jnp.int32),
            pltpu.VMEM((chunk, D), jnp.float32),
            pltpu.VMEM((chunk, D), jnp.float32),
            pltpu.SemaphoreType.DMA,
        ],
    )
    def gather_kernel(amp_hbm, phase_hbm, idx_hbm, amp_out, phase_out,
                      idx_v, amp_v, phase_v, sem):
        wid = lax.axis_index("s") * 2 + lax.axis_index("c")
        base0 = wid * b_per_w

        def body(i, carry):
            base = base0 + i * chunk
            pltpu.sync_copy(idx_hbm.at[pl.ds(base, chunk)], idx_v)
            c_amp = pltpu.async_copy(amp_hbm.at[idx_v], amp_v, sem)
            c_ph = pltpu.async_copy(phase_hbm.at[idx_v], phase_v, sem)
            c_amp.wait()
            c_ph.wait()
            pltpu.sync_copy(amp_v, amp_out.at[pl.ds(base, chunk)])
            pltpu.sync_copy(phase_v, phase_out.at[pl.ds(base, chunk)])
            return carry

        lax.fori_loop(0, n_chunks, body, 0)

    return gather_kernel


def kernel(amplitude_table, phase_table, indices):
    batch, hist = indices.shape
    d = amplitude_table.shape[1]
    b_total = batch * hist
    idx_flat = indices.reshape(b_total).astype(jnp.int32)
    k = _make_kernel(b_total, d, _CHUNK)
    amp, ph = k(amplitude_table, phase_table, idx_flat)
    return amp.reshape(batch, hist, d), ph.reshape(batch, hist, d)


# speedup vs baseline: 2.1207x; 2.1207x over previous
import functools

import jax
import jax.numpy as jnp
from jax import lax
from jax.experimental import pallas as pl
from jax.experimental.pallas import tpu as pltpu
from jax.experimental.pallas import tpu_sc as plsc

_NUM_WORKERS = 32
_CHUNK = 512


@functools.lru_cache(maxsize=None)
def _make_kernel(B, D, chunk):
    b_per_w = B // _NUM_WORKERS
    n_chunks = b_per_w // chunk
    mesh = plsc.VectorSubcoreMesh(core_axis_name="c", subcore_axis_name="s")

    @functools.partial(
        pl.kernel,
        mesh=mesh,
        out_type=(
            jax.ShapeDtypeStruct((B, D), jnp.float32),
            jax.ShapeDtypeStruct((B, D), jnp.float32),
        ),
        scratch_types=[
            pltpu.VMEM((chunk,), jnp.int32),
            pltpu.VMEM((chunk, D), jnp.float32),
            pltpu.VMEM((chunk, D), jnp.float32),
            pltpu.SemaphoreType.DMA,
        ],
        compiler_params=pltpu.CompilerParams(use_tc_tiling_on_sc=False),
    )
    def gather_kernel(amp_hbm, phase_hbm, idx_hbm, amp_out, phase_out,
                      idx_v, amp_v, phase_v, sem):
        wid = lax.axis_index("s") * 2 + lax.axis_index("c")
        base0 = wid * b_per_w

        def body(i, carry):
            base = base0 + i * chunk
            pltpu.sync_copy(idx_hbm.at[pl.ds(base, chunk)], idx_v)
            c_amp = pltpu.async_copy(amp_hbm.at[idx_v], amp_v, sem)
            c_ph = pltpu.async_copy(phase_hbm.at[idx_v], phase_v, sem)
            c_amp.wait()
            c_ph.wait()
            pltpu.sync_copy(amp_v, amp_out.at[pl.ds(base, chunk)])
            pltpu.sync_copy(phase_v, phase_out.at[pl.ds(base, chunk)])
            return carry

        lax.fori_loop(0, n_chunks, body, 0)

    return gather_kernel


def kernel(amplitude_table, phase_table, indices):
    batch, hist = indices.shape
    d = amplitude_table.shape[1]
    b_total = batch * hist
    idx_flat = indices.reshape(b_total).astype(jnp.int32)
    k = _make_kernel(b_total, d, _CHUNK)
    amp, ph = k(amplitude_table, phase_table, idx_flat)
    return amp.reshape(batch, hist, d), ph.reshape(batch, hist, d)


# speedup vs baseline: 2.1687x; 1.0226x over previous
import functools

import jax
import jax.numpy as jnp
from jax import lax
from jax.experimental import pallas as pl
from jax.experimental.pallas import tpu as pltpu
from jax.experimental.pallas import tpu_sc as plsc

_NUM_WORKERS = 32
_CHUNK = 320
_NBUF = 2


@functools.lru_cache(maxsize=None)
def _make_kernel(B, D, chunk):
    b_per_w = B // _NUM_WORKERS
    n_chunks = b_per_w // chunk
    n_outer = n_chunks // _NBUF
    mesh = plsc.VectorSubcoreMesh(core_axis_name="c", subcore_axis_name="s")

    @functools.partial(
        pl.kernel,
        mesh=mesh,
        out_type=(
            jax.ShapeDtypeStruct((B, D), jnp.float32),
            jax.ShapeDtypeStruct((B, D), jnp.float32),
        ),
        scratch_types=[
            pltpu.VMEM((b_per_w,), jnp.int32),
            pltpu.VMEM((_NBUF, chunk, D), jnp.float32),
            pltpu.VMEM((_NBUF, chunk, D), jnp.float32),
            pltpu.SemaphoreType.DMA((_NBUF,)),
        ],
        compiler_params=pltpu.CompilerParams(use_tc_tiling_on_sc=False),
    )
    def gather_kernel(amp_hbm, phase_hbm, idx_hbm, amp_out, phase_out,
                      idx_v, amp_v, phase_v, gsem):
        wid = lax.axis_index("s") * 2 + lax.axis_index("c")
        base0 = wid * b_per_w
        pltpu.sync_copy(idx_hbm.at[pl.ds(base0, b_per_w)], idx_v)

        def fire(r, b):
            idx_slice = idx_v.at[pl.ds(r * chunk, chunk)]
            pltpu.async_copy(amp_hbm.at[idx_slice], amp_v.at[b], gsem.at[b])
            pltpu.async_copy(phase_hbm.at[idx_slice], phase_v.at[b], gsem.at[b])

        def drain(r, b):
            idx_slice = idx_v.at[pl.ds(r * chunk, chunk)]
            pltpu.make_async_copy(amp_hbm.at[idx_slice], amp_v.at[b],
                                  gsem.at[b]).wait()
            pltpu.make_async_copy(phase_hbm.at[idx_slice], phase_v.at[b],
                                  gsem.at[b]).wait()

        fire(0, 0)

        def body(g, carry):
            for b in range(_NBUF):
                r = g * _NBUF + b
                nb = (b + 1) % _NBUF
                if b < _NBUF - 1:
                    fire(r + 1, nb)
                else:
                    @pl.when(g < n_outer - 1)
                    def _():
                        fire(r + 1, nb)
                drain(r, b)
                out_base = base0 + r * chunk
                pltpu.sync_copy(amp_v.at[b], amp_out.at[pl.ds(out_base, chunk)])
                pltpu.sync_copy(phase_v.at[b],
                                phase_out.at[pl.ds(out_base, chunk)])
            return carry

        lax.fori_loop(0, n_outer, body, 0)

    return gather_kernel


def kernel(amplitude_table, phase_table, indices):
    batch, hist = indices.shape
    d = amplitude_table.shape[1]
    b_total = batch * hist
    idx_flat = indices.reshape(b_total).astype(jnp.int32)
    k = _make_kernel(b_total, d, _CHUNK)
    amp, ph = k(amplitude_table, phase_table, idx_flat)
    return amp.reshape(batch, hist, d), ph.reshape(batch, hist, d)
